# Initial kernel scaffold; baseline (speedup 1.0000x reference)
import jax
import jax.numpy as jnp
from jax.experimental import pallas as pl


def kernel(inputs, table):
    raise NotImplementedError("write your pallas kernel here")


# speedup vs baseline: 136.8438x; 136.8438x over previous
import functools

import jax
import jax.numpy as jnp
from jax import lax
from jax.experimental import pallas as pl
from jax.experimental.pallas import tpu as pltpu
from jax.experimental.pallas import tpu_sc as plsc

_DEFAULT_VALUE = 0

_info = plsc.get_sparse_core_info()
_NC = _info.num_cores
_NS = _info.num_subcores
_L = _info.num_lanes
_NW = _NC * _NS


def kernel(inputs, table):
    batch, seq = inputs.shape
    vocab = table.shape[0]
    n = batch * seq
    assert n % (_NW * _L) == 0
    per_w = n // _NW
    flat = inputs.reshape(n)

    mesh = plsc.VectorSubcoreMesh(core_axis_name="c", subcore_axis_name="s")

    @functools.partial(
        pl.kernel,
        mesh=mesh,
        out_type=jax.ShapeDtypeStruct((n,), jnp.int32),
        scratch_types=[
            pltpu.VMEM((vocab,), jnp.int32),
            pltpu.VMEM((per_w,), jnp.int32),
            pltpu.SemaphoreType.DMA,
            pltpu.SemaphoreType.DMA,
        ],
        compiler_params=pltpu.CompilerParams(needs_layout_passes=False),
    )
    def _lookup(idx_hbm, table_hbm, out_hbm, table_v, buf, sem_t, sem_i):
        wid = lax.axis_index("s") * _NC + lax.axis_index("c")
        base = wid * per_w
        cp_t = pltpu.async_copy(table_hbm, table_v, sem_t)
        cp_i = pltpu.async_copy(idx_hbm.at[pl.ds(base, per_w)], buf, sem_i)
        cp_i.wait()
        cp_t.wait()

        def body(i, carry):
            off = i * _L
            keys = buf[pl.ds(off, _L)]
            valid = (keys >= 0) & (keys < vocab)
            safe = jnp.clip(keys, 0, vocab - 1)
            vals = plsc.load_gather(table_v, [safe])
            buf[pl.ds(off, _L)] = jnp.where(
                valid, vals, jnp.int32(_DEFAULT_VALUE))
            return carry

        lax.fori_loop(0, per_w // _L, body, 0)
        pltpu.sync_copy(buf, out_hbm.at[pl.ds(base, per_w)])

    out = _lookup(flat, table)
    return out.reshape(batch, seq)


# speedup vs baseline: 177.9921x; 1.3007x over previous
import functools

import jax
import jax.numpy as jnp
from jax import lax
from jax.experimental import pallas as pl
from jax.experimental.pallas import tpu as pltpu
from jax.experimental.pallas import tpu_sc as plsc

_DEFAULT_VALUE = 0

_info = plsc.get_sparse_core_info()
_NC = _info.num_cores
_NS = _info.num_subcores
_L = _info.num_lanes
_NW = _NC * _NS


def kernel(inputs, table):
    batch, seq = inputs.shape
    vocab = table.shape[0]
    n = batch * seq
    assert n % (_NW * _L) == 0
    per_w = n // _NW
    flat = inputs.reshape(n)

    mesh = plsc.VectorSubcoreMesh(core_axis_name="c", subcore_axis_name="s")

    @functools.partial(
        pl.kernel,
        mesh=mesh,
        out_type=jax.ShapeDtypeStruct((n,), jnp.int32),
        scratch_types=[
            pltpu.VMEM((vocab,), jnp.int32),
            pltpu.VMEM((per_w,), jnp.int32),
            pltpu.SemaphoreType.DMA,
            pltpu.SemaphoreType.DMA,
        ],
        compiler_params=pltpu.CompilerParams(needs_layout_passes=False),
    )
    def _lookup(idx_hbm, table_hbm, out_hbm, table_v, buf, sem_t, sem_i):
        wid = lax.axis_index("s") * _NC + lax.axis_index("c")
        base = wid * per_w
        cp_t = pltpu.async_copy(table_hbm, table_v, sem_t)
        cp_i = pltpu.async_copy(idx_hbm.at[pl.ds(base, per_w)], buf, sem_i)
        cp_i.wait()
        cp_t.wait()

        @plsc.parallel_loop(0, per_w, step=_L, unroll=8)
        def body(off):
            keys = buf[pl.ds(off, _L)]
            valid = (keys >= 0) & (keys < vocab)
            safe = jnp.clip(keys, 0, vocab - 1)
            vals = plsc.load_gather(table_v, [safe])
            buf[pl.ds(off, _L)] = jnp.where(
                valid, vals, jnp.int32(_DEFAULT_VALUE))
        pltpu.sync_copy(buf, out_hbm.at[pl.ds(base, per_w)])

    out = _lookup(flat, table)
    return out.reshape(batch, seq)
